# baseline (speedup 1.0000x reference)
import functools

import jax
import jax.numpy as jnp
from jax import lax
from jax.experimental import pallas as pl
from jax.experimental.pallas import tpu as pltpu
from jax.experimental.pallas import tpu_sc as plsc

N_NODES = 10000
NPAD = 10240
E_EDGES = 160000
E_REAL = E_EDGES + N_NODES
HEADS = 2

NC, NS, L = 2, 16, 16
NW = NC * NS
EP = 172032
ET = EP // NW
SUMN = 2 * NPAD

_MESH = plsc.VectorSubcoreMesh(core_axis_name="c", subcore_axis_name="s")


def _wid():
    return lax.axis_index("s") * NC + lax.axis_index("c")


def _iota16():
    return lax.broadcasted_iota(jnp.int32, (L,), 0)


def _proj_body(x_ref, w_ref, amat_ref, xp_ref, a8_ref):
    xb = x_ref[...]
    xp = jnp.dot(xb, w_ref[...], preferred_element_type=jnp.float32)
    xp_ref[...] = xp
    a8_ref[...] = jnp.dot(xp, amat_ref[...], preferred_element_type=jnp.float32)


def _project(x, W, att_src, att_dst):
    N, Cin = x.shape
    F = W.shape[1]
    H, C = att_src.shape
    z = jnp.zeros((C,), jnp.float32)
    cols = [
        jnp.concatenate([att_src[0], z]),
        jnp.concatenate([z, att_src[1]]),
        jnp.concatenate([att_dst[0], z]),
        jnp.concatenate([z, att_dst[1]]),
    ]
    amat = jnp.stack(cols + [jnp.zeros((F,), jnp.float32)] * 4, axis=1)
    BN = 1000
    xp, a8 = pl.pallas_call(
        _proj_body,
        grid=(N // BN,),
        in_specs=[
            pl.BlockSpec((BN, Cin), lambda i: (i, 0)),
            pl.BlockSpec((Cin, F), lambda i: (0, 0)),
            pl.BlockSpec((F, 8), lambda i: (0, 0)),
        ],
        out_specs=[
            pl.BlockSpec((BN, F), lambda i: (i, 0)),
            pl.BlockSpec((BN, 8), lambda i: (i, 0)),
        ],
        out_shape=[
            jax.ShapeDtypeStruct((N, F), jnp.float32),
            jax.ShapeDtypeStruct((N, 8), jnp.float32),
        ],
    )(x, W, amat)
    return xp, a8


@functools.partial(
    pl.kernel,
    out_type=[
        jax.ShapeDtypeStruct((EP,), jnp.float32),
        jax.ShapeDtypeStruct((EP,), jnp.float32),
    ],
    mesh=_MESH,
    compiler_params=pltpu.CompilerParams(needs_layout_passes=False),
    scratch_types=[
        pltpu.VMEM((4 * N_NODES,), jnp.float32),
        pltpu.VMEM((ET,), jnp.int32),
        pltpu.VMEM((ET,), jnp.int32),
        pltpu.VMEM((ET,), jnp.float32),
        pltpu.VMEM((ET,), jnp.float32),
        pltpu.VMEM((2 * L,), jnp.float32),
    ],
)
def _sc_edge_g(src_hbm, dst_hbm, a4_hbm, kv_hbm, g0_hbm, g1_hbm,
               a_tab, src_v, dst_v, g0_v, g1_v, kv_v):
    w = _wid()
    base = w * ET
    pltpu.sync_copy(a4_hbm, a_tab)
    pltpu.sync_copy(kv_hbm, kv_v)
    pltpu.sync_copy(src_hbm.at[pl.ds(base, ET)], src_v)
    pltpu.sync_copy(dst_hbm.at[pl.ds(base, ET)], dst_v)
    k0 = kv_v[pl.ds(0, L)]
    k1 = kv_v[pl.ds(L, L)]
    iota = _iota16()

    def body(i, _):
        off = i * L
        s16 = src_v[pl.ds(off, L)]
        d16 = dst_v[pl.ds(off, L)]
        as0 = plsc.load_gather(a_tab, [s16])
        as1 = plsc.load_gather(a_tab, [s16 + N_NODES])
        ad0 = plsc.load_gather(a_tab, [d16 + 2 * N_NODES])
        ad1 = plsc.load_gather(a_tab, [d16 + 3 * N_NODES])
        m = (base + off + iota) < E_REAL
        e0 = as0 + ad0
        e0 = jnp.where(e0 >= 0.0, e0, 0.2 * e0)
        g0 = jnp.where(m, jnp.exp(e0 - k0), 0.0)
        e1 = as1 + ad1
        e1 = jnp.where(e1 >= 0.0, e1, 0.2 * e1)
        g1 = jnp.where(m, jnp.exp(e1 - k1), 0.0)
        g0_v[pl.ds(off, L)] = g0
        g1_v[pl.ds(off, L)] = g1
        return _

    lax.fori_loop(0, ET // L, body, 0)
    pltpu.sync_copy(g0_v, g0_hbm.at[pl.ds(base, ET)])
    pltpu.sync_copy(g1_v, g1_hbm.at[pl.ds(base, ET)])


_A2_CH = 2048
_NPT = NPAD // NW


@functools.partial(
    pl.kernel,
    out_type=jax.ShapeDtypeStruct((SUMN,), jnp.float32),
    mesh=_MESH,
    compiler_params=pltpu.CompilerParams(needs_layout_passes=False),
    scratch_types=[
        pltpu.VMEM((_A2_CH,), jnp.int32),
        pltpu.VMEM((_A2_CH,), jnp.float32),
        pltpu.VMEM((_A2_CH,), jnp.float32),
        pltpu.VMEM((_NPT,), jnp.float32),
        pltpu.VMEM((_NPT,), jnp.float32),
    ],
)
def _sc_edge_sum(dst_hbm, g0_hbm, g1_hbm, asum_hbm,
                 dst_c, g0_c, g1_c, s0_v, s1_v):
    w = _wid()
    lo = w * _NPT
    zero = jnp.zeros((L,), jnp.float32)

    def zbody(i, _):
        s0_v[pl.ds(i * L, L)] = zero
        s1_v[pl.ds(i * L, L)] = zero
        return _

    lax.fori_loop(0, _NPT // L, zbody, 0)

    def chunk_body(ci, _):
        cbase = ci * _A2_CH
        pltpu.sync_copy(dst_hbm.at[pl.ds(cbase, _A2_CH)], dst_c)
        pltpu.sync_copy(g0_hbm.at[pl.ds(cbase, _A2_CH)], g0_c)
        pltpu.sync_copy(g1_hbm.at[pl.ds(cbase, _A2_CH)], g1_c)

        def vbody(i, _):
            off = i * L
            d16 = dst_c[pl.ds(off, L)]
            m = (d16 >= lo) & (d16 < lo + _NPT)
            idx = jnp.clip(d16 - lo, 0, _NPT - 1)
            plsc.addupdate_scatter(s0_v, [idx], g0_c[pl.ds(off, L)], mask=m)
            plsc.addupdate_scatter(s1_v, [idx], g1_c[pl.ds(off, L)], mask=m)
            return _

        lax.fori_loop(0, _A2_CH // L, vbody, 0)
        return _

    lax.fori_loop(0, EP // _A2_CH, chunk_body, 0)
    pltpu.sync_copy(s0_v, asum_hbm.at[pl.ds(lo, _NPT)])
    pltpu.sync_copy(s1_v, asum_hbm.at[pl.ds(NPAD + lo, _NPT)])


@functools.partial(
    pl.kernel,
    out_type=[
        jax.ShapeDtypeStruct((EP,), jnp.float32),
        jax.ShapeDtypeStruct((EP,), jnp.float32),
    ],
    mesh=_MESH,
    compiler_params=pltpu.CompilerParams(needs_layout_passes=False),
    scratch_types=[
        pltpu.VMEM((SUMN,), jnp.float32),
        pltpu.VMEM((ET,), jnp.int32),
        pltpu.VMEM((ET,), jnp.float32),
        pltpu.VMEM((ET,), jnp.float32),
    ],
)
def _sc_edge_alpha(dst_hbm, g0_hbm, g1_hbm, asum_hbm, al0_hbm, al1_hbm,
                   sum_v, dst_v, g0_v, g1_v):
    w = _wid()
    base = w * ET
    pltpu.sync_copy(asum_hbm, sum_v)
    pltpu.sync_copy(dst_hbm.at[pl.ds(base, ET)], dst_v)
    pltpu.sync_copy(g0_hbm.at[pl.ds(base, ET)], g0_v)
    pltpu.sync_copy(g1_hbm.at[pl.ds(base, ET)], g1_v)

    def body(i, _):
        off = i * L
        d16 = dst_v[pl.ds(off, L)]
        s0 = plsc.load_gather(sum_v, [d16])
        s1 = plsc.load_gather(sum_v, [d16 + NPAD])
        g0_v[pl.ds(off, L)] = g0_v[pl.ds(off, L)] / (s0 + 1e-16)
        g1_v[pl.ds(off, L)] = g1_v[pl.ds(off, L)] / (s1 + 1e-16)
        return _

    lax.fori_loop(0, ET // L, body, 0)
    pltpu.sync_copy(g0_v, al0_hbm.at[pl.ds(base, ET)])
    pltpu.sync_copy(g1_v, al1_hbm.at[pl.ds(base, ET)])


F_DIM = 512
ROWS_PT = 160
NPASS = 2
NOUT = ROWS_PT * NW * NPASS
SCAN = 1536
NCH = EP // SCAN
GB = 48


def _make_sc_aggregate(relu):
    @functools.partial(
        pl.kernel,
        out_type=jax.ShapeDtypeStruct((NOUT, F_DIM), jnp.float32),
        mesh=_MESH,
        compiler_params=pltpu.CompilerParams(needs_layout_passes=False),
        scratch_types=[
            pltpu.VMEM((2, SCAN), jnp.int32),
            pltpu.VMEM((2, SCAN), jnp.int32),
            pltpu.VMEM((2, SCAN), jnp.float32),
            pltpu.VMEM((2, SCAN), jnp.float32),
            pltpu.VMEM((SCAN + 2 * GB,), jnp.int32),
            pltpu.VMEM((SCAN + 2 * GB,), jnp.int32),
            pltpu.VMEM((SCAN + 2 * GB,), jnp.float32),
            pltpu.VMEM((SCAN + 2 * GB,), jnp.float32),
            pltpu.VMEM((GB, F_DIM), jnp.float32),
            pltpu.VMEM((ROWS_PT, F_DIM), jnp.float32),
            pltpu.VMEM((F_DIM,), jnp.float32),
            pltpu.SemaphoreType.DMA((4, 2)),
            pltpu.SemaphoreType.DMA,
        ],
    )
    def _sc_aggregate(src_hbm, dst_hbm, al0_hbm, al1_hbm, xp_hbm, b_hbm,
                      out_hbm, e_src, e_dst, e_a0, e_a1, c_src, c_dst, c_a0,
                      c_a1, rowbuf, acc, bias_v, sems, gsem):
        w = _wid()
        pltpu.sync_copy(b_hbm, bias_v)
        zero = jnp.zeros((L,), jnp.float32)
        izero = jnp.zeros((L,), jnp.int32)

        def _fire(ci, slot):
            eb = ci * SCAN
            pltpu.async_copy(src_hbm.at[pl.ds(eb, SCAN)], e_src.at[slot],
                             sems.at[0, slot])
            pltpu.async_copy(dst_hbm.at[pl.ds(eb, SCAN)], e_dst.at[slot],
                             sems.at[1, slot])
            pltpu.async_copy(al0_hbm.at[pl.ds(eb, SCAN)], e_a0.at[slot],
                             sems.at[2, slot])
            pltpu.async_copy(al1_hbm.at[pl.ds(eb, SCAN)], e_a1.at[slot],
                             sems.at[3, slot])

        def _wait(ci, slot):
            eb = ci * SCAN
            pltpu.make_async_copy(src_hbm.at[pl.ds(eb, SCAN)], e_src.at[slot],
                                  sems.at[0, slot]).wait()
            pltpu.make_async_copy(dst_hbm.at[pl.ds(eb, SCAN)], e_dst.at[slot],
                                  sems.at[1, slot]).wait()
            pltpu.make_async_copy(al0_hbm.at[pl.ds(eb, SCAN)], e_a0.at[slot],
                                  sems.at[2, slot]).wait()
            pltpu.make_async_copy(al1_hbm.at[pl.ds(eb, SCAN)], e_a1.at[slot],
                                  sems.at[3, slot]).wait()

        for p in range(NPASS):
            row_lo = (p * NW + w) * ROWS_PT

            def zb(i, _):
                acc[i // 32, pl.ds((i % 32) * L, L)] = zero
                return _

            lax.fori_loop(0, ROWS_PT * 32, zb, 0)
            _fire(0, 0)

            def _process(ci, slot):
                def scan(i, cnt):
                    off = i * L
                    d16 = e_dst[slot, pl.ds(off, L)]
                    m = (d16 >= row_lo) & (d16 < row_lo + ROWS_PT)
                    plsc.store_compressed(c_src.at[pl.ds(cnt, L)],
                                          e_src[slot, pl.ds(off, L)], mask=m)
                    plsc.store_compressed(c_dst.at[pl.ds(cnt, L)],
                                          d16 - row_lo, mask=m)
                    plsc.store_compressed(c_a0.at[pl.ds(cnt, L)],
                                          e_a0[slot, pl.ds(off, L)], mask=m)
                    plsc.store_compressed(c_a1.at[pl.ds(cnt, L)],
                                          e_a1[slot, pl.ds(off, L)], mask=m)
                    pc = plsc.all_reduce_population_count(m)
                    return cnt + pc[0]

                cnt = lax.fori_loop(0, SCAN // L, scan, 0)
                for t in range(GB // L):
                    c_src[pl.ds(cnt + t * L, L)] = izero
                    c_dst[pl.ds(cnt + t * L, L)] = izero
                    c_a0[pl.ds(cnt + t * L, L)] = zero
                    c_a1[pl.ds(cnt + t * L, L)] = zero
                nb = (cnt + GB - 1) // GB

                def gbody(b, gc):
                    gb0 = b * GB
                    pltpu.async_copy(xp_hbm.at[c_src.at[pl.ds(gb0, GB)]],
                                     rowbuf, gsem).wait()

                    def rbody(j, __):
                        splat = izero + (gb0 + j)
                        a0s = plsc.load_gather(c_a0, [splat])
                        a1s = plsc.load_gather(c_a1, [splat])
                        dj = plsc.load_gather(c_dst, [splat])[0]
                        for q in range(16):
                            plsc.addupdate(acc.at[dj, pl.ds(q * L, L)],
                                           rowbuf[j, pl.ds(q * L, L)] * a0s)
                        for q in range(16, 32):
                            plsc.addupdate(acc.at[dj, pl.ds(q * L, L)],
                                           rowbuf[j, pl.ds(q * L, L)] * a1s)
                        return __

                    lax.fori_loop(0, GB, rbody, 0)
                    return gc

                lax.fori_loop(0, nb, gbody, 0)

            def chunk_pair(i2, carry):
                c0 = i2 * 2

                @pl.when(c0 + 1 < NCH)
                def _f1():
                    _fire(c0 + 1, 1)

                _wait(c0, 0)
                _process(c0, 0)

                @pl.when(c0 + 2 < NCH)
                def _f2():
                    _fire(c0 + 2, 0)

                _wait(c0 + 1, 1)
                _process(c0 + 1, 1)
                return carry

            lax.fori_loop(0, NCH // 2, chunk_pair, 0)

            def obody(r, _):
                for q in range(32):
                    v = acc[r, pl.ds(q * L, L)] + bias_v[pl.ds(q * L, L)]
                    if relu:
                        v = jnp.maximum(v, 0.0)
                    acc[r, pl.ds(q * L, L)] = v
                return _

            lax.fori_loop(0, ROWS_PT, obody, 0)
            pltpu.sync_copy(acc, out_hbm.at[pl.ds(row_lo, ROWS_PT)])

    return _sc_aggregate


_sc_aggregate_relu = _make_sc_aggregate(True)
_sc_aggregate_plain = _make_sc_aggregate(False)


def _edge_softmax(a8, srcp, dstp):
    a_s = a8[:, 0:2]
    a_d = a8[:, 2:4]
    K = jax.nn.leaky_relu(jnp.max(a_s, axis=0) + jnp.max(a_d, axis=0), 0.2)
    a4 = jnp.concatenate([a8[:, 0], a8[:, 1], a8[:, 2], a8[:, 3]])
    kv = jnp.concatenate([jnp.full((L,), K[0]), jnp.full((L,), K[1])])
    g0, g1 = _sc_edge_g(srcp, dstp, a4, kv)
    asum = _sc_edge_sum(dstp, g0, g1)
    al0, al1 = _sc_edge_alpha(dstp, g0, g1, asum)
    return al0, al1


def _edge_stage(xp, a8, srcp, dstp, n_nodes, b, relu):
    al0, al1 = _edge_softmax(a8, srcp, dstp)
    alpha = jnp.stack([al0[:E_REAL], al1[:E_REAL]], axis=1)
    agg = _sc_aggregate_relu if relu else _sc_aggregate_plain
    h = agg(srcp, dstp, al0, al1, xp, b)
    return h[:n_nodes], alpha


def kernel(x, edge_index, W1, att_src1, att_dst1, b1, W2, att_src2, att_dst2, b2):
    src = edge_index[0].astype(jnp.int32)
    dst = edge_index[1].astype(jnp.int32)
    loop = jnp.arange(N_NODES, dtype=jnp.int32)
    pad = jnp.zeros((EP - E_REAL,), jnp.int32)
    srcp = jnp.concatenate([src, loop, pad])
    dstp = jnp.concatenate([dst, loop, pad])

    xp1, a8_1 = _project(x, W1, att_src1, att_dst1)
    h1, alpha1 = _edge_stage(xp1, a8_1, srcp, dstp, N_NODES, b1, relu=True)

    xp2, a8_2 = _project(h1, W2, att_src2, att_dst2)
    h2, alpha2 = _edge_stage(xp2, a8_2, srcp, dstp, N_NODES, b2, relu=False)
    return (h2, alpha1, alpha2)


# speedup vs baseline: 1.9263x; 1.9263x over previous
import functools

import jax
import jax.numpy as jnp
from jax import lax
from jax.experimental import pallas as pl
from jax.experimental.pallas import tpu as pltpu
from jax.experimental.pallas import tpu_sc as plsc

N_NODES = 10000
NPAD = 10240
E_EDGES = 160000
E_REAL = E_EDGES + N_NODES
HEADS = 2

NC, NS, L = 2, 16, 16
NW = NC * NS
EP = 172032
ET = EP // NW
SUMN = 2 * NPAD

_MESH = plsc.VectorSubcoreMesh(core_axis_name="c", subcore_axis_name="s")


def _wid():
    return lax.axis_index("s") * NC + lax.axis_index("c")


def _iota16():
    return lax.broadcasted_iota(jnp.int32, (L,), 0)


def _proj_body(x_ref, w_ref, amat_ref, xp_ref, a8_ref):
    xb = x_ref[...]
    xp = jnp.dot(xb, w_ref[...], preferred_element_type=jnp.float32)
    xp_ref[...] = xp
    a8_ref[...] = jnp.dot(xp, amat_ref[...], preferred_element_type=jnp.float32)


def _project(x, W, att_src, att_dst):
    N, Cin = x.shape
    F = W.shape[1]
    H, C = att_src.shape
    z = jnp.zeros((C,), jnp.float32)
    cols = [
        jnp.concatenate([att_src[0], z]),
        jnp.concatenate([z, att_src[1]]),
        jnp.concatenate([att_dst[0], z]),
        jnp.concatenate([z, att_dst[1]]),
    ]
    amat = jnp.stack(cols + [jnp.zeros((F,), jnp.float32)] * 4, axis=1)
    BN = 1000
    xp, a8 = pl.pallas_call(
        _proj_body,
        grid=(N // BN,),
        in_specs=[
            pl.BlockSpec((BN, Cin), lambda i: (i, 0)),
            pl.BlockSpec((Cin, F), lambda i: (0, 0)),
            pl.BlockSpec((F, 8), lambda i: (0, 0)),
        ],
        out_specs=[
            pl.BlockSpec((BN, F), lambda i: (i, 0)),
            pl.BlockSpec((BN, 8), lambda i: (i, 0)),
        ],
        out_shape=[
            jax.ShapeDtypeStruct((N, F), jnp.float32),
            jax.ShapeDtypeStruct((N, 8), jnp.float32),
        ],
    )(x, W, amat)
    return xp, a8


@functools.partial(
    pl.kernel,
    out_type=[
        jax.ShapeDtypeStruct((EP,), jnp.float32),
        jax.ShapeDtypeStruct((EP,), jnp.float32),
    ],
    mesh=_MESH,
    compiler_params=pltpu.CompilerParams(needs_layout_passes=False),
    scratch_types=[
        pltpu.VMEM((4 * N_NODES,), jnp.float32),
        pltpu.VMEM((ET,), jnp.int32),
        pltpu.VMEM((ET,), jnp.int32),
        pltpu.VMEM((ET,), jnp.float32),
        pltpu.VMEM((ET,), jnp.float32),
        pltpu.VMEM((2 * L,), jnp.float32),
    ],
)
def _sc_edge_g(src_hbm, dst_hbm, a4_hbm, kv_hbm, g0_hbm, g1_hbm,
               a_tab, src_v, dst_v, g0_v, g1_v, kv_v):
    w = _wid()
    base = w * ET
    pltpu.sync_copy(a4_hbm, a_tab)
    pltpu.sync_copy(kv_hbm, kv_v)
    pltpu.sync_copy(src_hbm.at[pl.ds(base, ET)], src_v)
    pltpu.sync_copy(dst_hbm.at[pl.ds(base, ET)], dst_v)
    k0 = kv_v[pl.ds(0, L)]
    k1 = kv_v[pl.ds(L, L)]
    iota = _iota16()

    def body(i, _):
        off = i * L
        s16 = src_v[pl.ds(off, L)]
        d16 = dst_v[pl.ds(off, L)]
        as0 = plsc.load_gather(a_tab, [s16])
        as1 = plsc.load_gather(a_tab, [s16 + N_NODES])
        ad0 = plsc.load_gather(a_tab, [d16 + 2 * N_NODES])
        ad1 = plsc.load_gather(a_tab, [d16 + 3 * N_NODES])
        m = (base + off + iota) < E_REAL
        e0 = as0 + ad0
        e0 = jnp.where(e0 >= 0.0, e0, 0.2 * e0)
        g0 = jnp.where(m, jnp.exp(e0 - k0), 0.0)
        e1 = as1 + ad1
        e1 = jnp.where(e1 >= 0.0, e1, 0.2 * e1)
        g1 = jnp.where(m, jnp.exp(e1 - k1), 0.0)
        g0_v[pl.ds(off, L)] = g0
        g1_v[pl.ds(off, L)] = g1
        return _

    lax.fori_loop(0, ET // L, body, 0)
    pltpu.sync_copy(g0_v, g0_hbm.at[pl.ds(base, ET)])
    pltpu.sync_copy(g1_v, g1_hbm.at[pl.ds(base, ET)])


_A2_CH = 2048
_NPT = NPAD // NW


@functools.partial(
    pl.kernel,
    out_type=jax.ShapeDtypeStruct((SUMN,), jnp.float32),
    mesh=_MESH,
    compiler_params=pltpu.CompilerParams(needs_layout_passes=False),
    scratch_types=[
        pltpu.VMEM((_A2_CH,), jnp.int32),
        pltpu.VMEM((_A2_CH,), jnp.float32),
        pltpu.VMEM((_A2_CH,), jnp.float32),
        pltpu.VMEM((_NPT,), jnp.float32),
        pltpu.VMEM((_NPT,), jnp.float32),
    ],
)
def _sc_edge_sum(dst_hbm, g0_hbm, g1_hbm, asum_hbm,
                 dst_c, g0_c, g1_c, s0_v, s1_v):
    w = _wid()
    lo = w * _NPT
    zero = jnp.zeros((L,), jnp.float32)

    def zbody(i, _):
        s0_v[pl.ds(i * L, L)] = zero
        s1_v[pl.ds(i * L, L)] = zero
        return _

    lax.fori_loop(0, _NPT // L, zbody, 0)

    def chunk_body(ci, _):
        cbase = ci * _A2_CH
        pltpu.sync_copy(dst_hbm.at[pl.ds(cbase, _A2_CH)], dst_c)
        pltpu.sync_copy(g0_hbm.at[pl.ds(cbase, _A2_CH)], g0_c)
        pltpu.sync_copy(g1_hbm.at[pl.ds(cbase, _A2_CH)], g1_c)

        def vbody(i, _):
            off = i * L
            d16 = dst_c[pl.ds(off, L)]
            m = (d16 >= lo) & (d16 < lo + _NPT)
            idx = jnp.clip(d16 - lo, 0, _NPT - 1)
            plsc.addupdate_scatter(s0_v, [idx], g0_c[pl.ds(off, L)], mask=m)
            plsc.addupdate_scatter(s1_v, [idx], g1_c[pl.ds(off, L)], mask=m)
            return _

        lax.fori_loop(0, _A2_CH // L, vbody, 0)
        return _

    lax.fori_loop(0, EP // _A2_CH, chunk_body, 0)
    pltpu.sync_copy(s0_v, asum_hbm.at[pl.ds(lo, _NPT)])
    pltpu.sync_copy(s1_v, asum_hbm.at[pl.ds(NPAD + lo, _NPT)])


@functools.partial(
    pl.kernel,
    out_type=[
        jax.ShapeDtypeStruct((EP,), jnp.float32),
        jax.ShapeDtypeStruct((EP,), jnp.float32),
    ],
    mesh=_MESH,
    compiler_params=pltpu.CompilerParams(needs_layout_passes=False),
    scratch_types=[
        pltpu.VMEM((SUMN,), jnp.float32),
        pltpu.VMEM((ET,), jnp.int32),
        pltpu.VMEM((ET,), jnp.float32),
        pltpu.VMEM((ET,), jnp.float32),
    ],
)
def _sc_edge_alpha(dst_hbm, g0_hbm, g1_hbm, asum_hbm, al0_hbm, al1_hbm,
                   sum_v, dst_v, g0_v, g1_v):
    w = _wid()
    base = w * ET
    pltpu.sync_copy(asum_hbm, sum_v)
    pltpu.sync_copy(dst_hbm.at[pl.ds(base, ET)], dst_v)
    pltpu.sync_copy(g0_hbm.at[pl.ds(base, ET)], g0_v)
    pltpu.sync_copy(g1_hbm.at[pl.ds(base, ET)], g1_v)

    def body(i, _):
        off = i * L
        d16 = dst_v[pl.ds(off, L)]
        s0 = plsc.load_gather(sum_v, [d16])
        s1 = plsc.load_gather(sum_v, [d16 + NPAD])
        g0_v[pl.ds(off, L)] = g0_v[pl.ds(off, L)] / (s0 + 1e-16)
        g1_v[pl.ds(off, L)] = g1_v[pl.ds(off, L)] / (s1 + 1e-16)
        return _

    lax.fori_loop(0, ET // L, body, 0)
    pltpu.sync_copy(g0_v, al0_hbm.at[pl.ds(base, ET)])
    pltpu.sync_copy(g1_v, al1_hbm.at[pl.ds(base, ET)])


F_DIM = 512
ROWS_PT = 160
NPASS = 2
NOUT = ROWS_PT * NW * NPASS
SCAN = 1792
NCH = EP // SCAN
GB = 32


def _make_sc_aggregate(relu):
    @functools.partial(
        pl.kernel,
        out_type=jax.ShapeDtypeStruct((NOUT, F_DIM), jnp.float32),
        mesh=_MESH,
        compiler_params=pltpu.CompilerParams(needs_layout_passes=False),
        scratch_types=[
            pltpu.VMEM((2, SCAN), jnp.int32),
            pltpu.VMEM((2, SCAN), jnp.int32),
            pltpu.VMEM((2, SCAN), jnp.float32),
            pltpu.VMEM((2, SCAN), jnp.float32),
            pltpu.VMEM((SCAN + 2 * GB,), jnp.int32),
            pltpu.VMEM((SCAN + 2 * GB,), jnp.int32),
            pltpu.VMEM((SCAN + 2 * GB,), jnp.float32),
            pltpu.VMEM((SCAN + 2 * GB,), jnp.float32),
            pltpu.VMEM((GB, F_DIM), jnp.float32),
            pltpu.VMEM((ROWS_PT, F_DIM), jnp.float32),
            pltpu.VMEM((F_DIM,), jnp.float32),
            pltpu.SemaphoreType.DMA((4, 2)),
            pltpu.SemaphoreType.DMA,
        ],
    )
    def _sc_aggregate(src_hbm, dst_hbm, al0_hbm, al1_hbm, xp_hbm, b_hbm,
                      out_hbm, e_src, e_dst, e_a0, e_a1, c_src, c_dst, c_a0,
                      c_a1, rowbuf, acc, bias_v, sems, gsem):
        w = _wid()
        pltpu.sync_copy(b_hbm, bias_v)
        zero = jnp.zeros((L,), jnp.float32)
        izero = jnp.zeros((L,), jnp.int32)

        def _fire(ci, slot):
            eb = ci * SCAN
            pltpu.async_copy(src_hbm.at[pl.ds(eb, SCAN)], e_src.at[slot],
                             sems.at[0, slot])
            pltpu.async_copy(dst_hbm.at[pl.ds(eb, SCAN)], e_dst.at[slot],
                             sems.at[1, slot])
            pltpu.async_copy(al0_hbm.at[pl.ds(eb, SCAN)], e_a0.at[slot],
                             sems.at[2, slot])
            pltpu.async_copy(al1_hbm.at[pl.ds(eb, SCAN)], e_a1.at[slot],
                             sems.at[3, slot])

        def _wait(ci, slot):
            eb = ci * SCAN
            pltpu.make_async_copy(src_hbm.at[pl.ds(eb, SCAN)], e_src.at[slot],
                                  sems.at[0, slot]).wait()
            pltpu.make_async_copy(dst_hbm.at[pl.ds(eb, SCAN)], e_dst.at[slot],
                                  sems.at[1, slot]).wait()
            pltpu.make_async_copy(al0_hbm.at[pl.ds(eb, SCAN)], e_a0.at[slot],
                                  sems.at[2, slot]).wait()
            pltpu.make_async_copy(al1_hbm.at[pl.ds(eb, SCAN)], e_a1.at[slot],
                                  sems.at[3, slot]).wait()

        def pass_body(p, pc_):
            row_lo = (p * NW + w) * ROWS_PT

            def zb(i, _):
                acc[i // 32, pl.ds((i % 32) * L, L)] = zero
                return _

            lax.fori_loop(0, ROWS_PT * 32, zb, 0)
            _fire(0, 0)

            def _process(ci, slot):
                def scan(i, cnt):
                    off = i * L
                    d16 = e_dst[slot, pl.ds(off, L)]
                    m = (d16 >= row_lo) & (d16 < row_lo + ROWS_PT)
                    plsc.store_compressed(c_src.at[pl.ds(cnt, L)],
                                          e_src[slot, pl.ds(off, L)], mask=m)
                    plsc.store_compressed(c_dst.at[pl.ds(cnt, L)],
                                          d16 - row_lo, mask=m)
                    plsc.store_compressed(c_a0.at[pl.ds(cnt, L)],
                                          e_a0[slot, pl.ds(off, L)], mask=m)
                    plsc.store_compressed(c_a1.at[pl.ds(cnt, L)],
                                          e_a1[slot, pl.ds(off, L)], mask=m)
                    pc = plsc.all_reduce_population_count(m)
                    return cnt + pc[0]

                cnt = lax.fori_loop(0, SCAN // L, scan, 0)
                for t in range(GB // L):
                    c_src[pl.ds(cnt + t * L, L)] = izero
                    c_dst[pl.ds(cnt + t * L, L)] = izero
                    c_a0[pl.ds(cnt + t * L, L)] = zero
                    c_a1[pl.ds(cnt + t * L, L)] = zero
                nb = (cnt + GB - 1) // GB

                def gbody(b, gc):
                    gb0 = b * GB
                    pltpu.async_copy(xp_hbm.at[c_src.at[pl.ds(gb0, GB)]],
                                     rowbuf, gsem).wait()

                    def halfb(h, hc):
                        hb = gb0 + h * L
                        av0 = c_a0[pl.ds(hb, L)]
                        av1 = c_a1[pl.ds(hb, L)]
                        dv = c_dst[pl.ds(hb, L)]
                        for j in range(L):
                            a0s = av0[j]
                            a1s = av1[j]
                            dj = dv[j]
                            r = h * L + j

                            def qa_body(qa, qc):
                                for qi in range(8):
                                    q = qa * 8 + qi
                                    plsc.addupdate(
                                        acc.at[dj, pl.ds(q * L, L)],
                                        rowbuf[r, pl.ds(q * L, L)] * a0s)
                                return qc

                            lax.fori_loop(0, 2, qa_body, 0)

                            def qb_body(qb, qc):
                                for qi in range(8):
                                    q = 16 + qb * 8 + qi
                                    plsc.addupdate(
                                        acc.at[dj, pl.ds(q * L, L)],
                                        rowbuf[r, pl.ds(q * L, L)] * a1s)
                                return qc

                            lax.fori_loop(0, 2, qb_body, 0)
                        return hc

                    lax.fori_loop(0, GB // L, halfb, 0)
                    return gc

                lax.fori_loop(0, nb, gbody, 0)

            def chunk_pair(i2, carry):
                c0 = i2 * 2

                @pl.when(c0 + 1 < NCH)
                def _f1():
                    _fire(c0 + 1, 1)

                _wait(c0, 0)
                _process(c0, 0)

                @pl.when(c0 + 2 < NCH)
                def _f2():
                    _fire(c0 + 2, 0)

                _wait(c0 + 1, 1)
                _process(c0 + 1, 1)
                return carry

            lax.fori_loop(0, NCH // 2, chunk_pair, 0)

            def obody(r, _):
                def oq(qo, qc):
                    for qi in range(8):
                        q = qo * 8 + qi
                        v = acc[r, pl.ds(q * L, L)] + bias_v[pl.ds(q * L, L)]
                        if relu:
                            v = jnp.maximum(v, 0.0)
                        acc[r, pl.ds(q * L, L)] = v
                    return qc

                lax.fori_loop(0, 4, oq, 0)
                return _

            lax.fori_loop(0, ROWS_PT, obody, 0)
            pltpu.sync_copy(acc, out_hbm.at[pl.ds(row_lo, ROWS_PT)])
            return pc_

        lax.fori_loop(0, NPASS, pass_body, 0)

    return _sc_aggregate


_sc_aggregate_relu = _make_sc_aggregate(True)
_sc_aggregate_plain = _make_sc_aggregate(False)


def _edge_softmax(a8, srcp, dstp):
    a_s = a8[:, 0:2]
    a_d = a8[:, 2:4]
    K = jax.nn.leaky_relu(jnp.max(a_s, axis=0) + jnp.max(a_d, axis=0), 0.2)
    a4 = jnp.concatenate([a8[:, 0], a8[:, 1], a8[:, 2], a8[:, 3]])
    kv = jnp.concatenate([jnp.full((L,), K[0]), jnp.full((L,), K[1])])
    g0, g1 = _sc_edge_g(srcp, dstp, a4, kv)
    asum = _sc_edge_sum(dstp, g0, g1)
    al0, al1 = _sc_edge_alpha(dstp, g0, g1, asum)
    return al0, al1


def _edge_stage(xp, a8, srcp, dstp, n_nodes, b, relu):
    al0, al1 = _edge_softmax(a8, srcp, dstp)
    alpha = jnp.stack([al0[:E_REAL], al1[:E_REAL]], axis=1)
    agg = _sc_aggregate_relu if relu else _sc_aggregate_plain
    h = agg(srcp, dstp, al0, al1, xp, b)
    return h[:n_nodes], alpha


def kernel(x, edge_index, W1, att_src1, att_dst1, b1, W2, att_src2, att_dst2, b2):
    src = edge_index[0].astype(jnp.int32)
    dst = edge_index[1].astype(jnp.int32)
    loop = jnp.arange(N_NODES, dtype=jnp.int32)
    pad = jnp.zeros((EP - E_REAL,), jnp.int32)
    srcp = jnp.concatenate([src, loop, pad])
    dstp = jnp.concatenate([dst, loop, pad])

    xp1, a8_1 = _project(x, W1, att_src1, att_dst1)
    h1, alpha1 = _edge_stage(xp1, a8_1, srcp, dstp, N_NODES, b1, relu=True)

    xp2, a8_2 = _project(h1, W2, att_src2, att_dst2)
    h2, alpha2 = _edge_stage(xp2, a8_2, srcp, dstp, N_NODES, b2, relu=False)
    return (h2, alpha1, alpha2)
